# Initial kernel scaffold; baseline (speedup 1.0000x reference)
import jax
import jax.numpy as jnp
from jax.experimental import pallas as pl


def kernel(x, edge_index, W1, b1, g1, be1, W2, b2, g2, be2, W3, b3, g3, be3, Wout, bout):
    raise NotImplementedError("write your pallas kernel here")


# speedup vs baseline: 8.2569x; 8.2569x over previous
import functools

import jax
import jax.numpy as jnp
from jax import lax
from jax.experimental import pallas as pl
from jax.experimental.pallas import tpu as pltpu
from jax.experimental.pallas import tpu_sc as plsc

NC = 2
NS = 16
CHUNK = 128


def _mesh():
  return plsc.VectorSubcoreMesh(core_axis_name="c", subcore_axis_name="s")


def _make_deg_kernel(NP, EP):
  ch_per_tile = EP // (NC * NS * CHUNK)
  rows_per_tile = NP // NS

  @functools.partial(
      pl.kernel,
      out_type=jax.ShapeDtypeStruct((NC, NP, 128), jnp.float32),
      mesh=_mesh(),
      scratch_types=[
          pltpu.VMEM_SHARED((NP, 128), jnp.float32),
          pltpu.VMEM((CHUNK,), jnp.int32),
          pltpu.VMEM((CHUNK, 128), jnp.float32),
      ],
  )
  def deg_kernel(dst_hbm, ones_hbm, zeros_hbm, out_hbm, acc_sh, dst_v,
                 ones_v):
    c = lax.axis_index("c")
    s = lax.axis_index("s")
    pltpu.sync_copy(zeros_hbm,
                    acc_sh.at[pl.ds(s * rows_per_tile, rows_per_tile)])
    pltpu.sync_copy(ones_hbm, ones_v)
    plsc.subcore_barrier()

    wid = c * NS + s
    base0 = wid * (ch_per_tile * CHUNK)

    def body(j, carry):
      base = base0 + j * CHUNK
      pltpu.sync_copy(dst_hbm.at[pl.ds(base, CHUNK)], dst_v)
      pltpu.sync_copy(ones_v, acc_sh.at[dst_v], add=True)
      return carry

    lax.fori_loop(0, ch_per_tile, body, 0)
    plsc.subcore_barrier()
    pltpu.sync_copy(acc_sh.at[pl.ds(s * rows_per_tile, rows_per_tile)],
                    out_hbm.at[c].at[pl.ds(s * rows_per_tile, rows_per_tile)])

  return deg_kernel


def _make_agg_kernel(N, NP, EP, D):
  ch_per_tile = EP // (NC * NS * CHUNK)
  rows_per_tile = NP // NS

  @functools.partial(
      pl.kernel,
      out_type=jax.ShapeDtypeStruct((NC, NP, D), jnp.float32),
      mesh=_mesh(),
      scratch_types=[
          pltpu.VMEM_SHARED((NP, D), jnp.float32),
          pltpu.VMEM((CHUNK,), jnp.int32),
          pltpu.VMEM((CHUNK,), jnp.int32),
          pltpu.VMEM((CHUNK, D), jnp.float32),
          pltpu.SemaphoreType.DMA,
      ],
  )
  def agg_kernel(u_hbm, src_hbm, dst_hbm, zeros_hbm, out_hbm, acc_sh, src_v,
                 dst_v, rows_v, sem):
    c = lax.axis_index("c")
    s = lax.axis_index("s")
    pltpu.sync_copy(zeros_hbm,
                    acc_sh.at[pl.ds(s * rows_per_tile, rows_per_tile)])
    plsc.subcore_barrier()

    wid = c * NS + s
    base0 = wid * (ch_per_tile * CHUNK)

    def body(j, carry):
      base = base0 + j * CHUNK
      pltpu.sync_copy(src_hbm.at[pl.ds(base, CHUNK)], src_v)
      pltpu.sync_copy(dst_hbm.at[pl.ds(base, CHUNK)], dst_v)
      pltpu.async_copy(u_hbm.at[src_v], rows_v, sem).wait()
      pltpu.sync_copy(rows_v, acc_sh.at[dst_v], add=True)
      return carry

    lax.fori_loop(0, ch_per_tile, body, 0)
    plsc.subcore_barrier()
    pltpu.sync_copy(acc_sh.at[pl.ds(s * rows_per_tile, rows_per_tile)],
                    out_hbm.at[c].at[pl.ds(s * rows_per_tile, rows_per_tile)])

  return agg_kernel


def _tc_pre(deg2, x, W1, N):
  def body(d_ref, x_ref, w_ref, dinv_ref, u_ref):
    deg = d_ref[0, 0:N, 0:1] + d_ref[1, 0:N, 0:1] + 1.0
    dinv = lax.rsqrt(deg)
    dinv_ref[...] = dinv
    h = jnp.dot(x_ref[...], w_ref[...], preferred_element_type=jnp.float32)
    u_ref[...] = h * dinv

  return pl.pallas_call(
      body,
      out_shape=(
          jax.ShapeDtypeStruct((N, 1), jnp.float32),
          jax.ShapeDtypeStruct((N, x.shape[1]), jnp.float32),
      ),
  )(deg2, x, W1)


def _tc_layer(s2, u, dinv, b, g, be, Wn, N, D):
  def body(s_ref, u_ref, dinv_ref, b_ref, g_ref, be_ref, w_ref, out_ref):
    conv = (s_ref[0, 0:N, :] + s_ref[1, 0:N, :] + u_ref[...]) * dinv_ref[...]
    conv = conv + b_ref[...]
    mu = jnp.mean(conv, axis=0, keepdims=True)
    d = conv - mu
    var = jnp.mean(d * d, axis=0, keepdims=True)
    z = g_ref[...] * d * lax.rsqrt(var + 1e-5) + be_ref[...]
    z = jnp.maximum(z, 0.0)
    out_ref[...] = (
        jnp.dot(z, w_ref[...], preferred_element_type=jnp.float32)
        * dinv_ref[...])

  return pl.pallas_call(
      body,
      out_shape=jax.ShapeDtypeStruct((N, Wn.shape[1]), jnp.float32),
  )(s2, u, dinv, b.reshape(1, -1), g.reshape(1, -1), be.reshape(1, -1), Wn)


def _tc_final(s2, u, dinv, b, g, be, Wout, bout, N):
  def body(s_ref, u_ref, dinv_ref, b_ref, g_ref, be_ref, w_ref, bo_ref,
           out_ref):
    conv = (s_ref[0, 0:N, :] + s_ref[1, 0:N, :] + u_ref[...]) * dinv_ref[...]
    conv = conv + b_ref[...]
    mu = jnp.mean(conv, axis=0, keepdims=True)
    d = conv - mu
    var = jnp.mean(d * d, axis=0, keepdims=True)
    z = g_ref[...] * d * lax.rsqrt(var + 1e-5) + be_ref[...]
    z = jnp.maximum(z, 0.0)
    out_ref[...] = (
        jnp.dot(z, w_ref[...], preferred_element_type=jnp.float32)
        + bo_ref[...])

  return pl.pallas_call(
      body,
      out_shape=jax.ShapeDtypeStruct((N, Wout.shape[1]), jnp.float32),
  )(s2, u, dinv, b.reshape(1, -1), g.reshape(1, -1), be.reshape(1, -1), Wout,
    bout.reshape(1, -1))


def kernel(x, edge_index, W1, b1, g1, be1, W2, b2, g2, be2, W3, b3, g3, be3,
           Wout, bout):
  N, D = x.shape
  E = edge_index.shape[1]
  NP = ((N + 1 + NS * 8 - 1) // (NS * 8)) * (NS * 8)
  per_round = NC * NS * CHUNK
  EP = ((E + per_round - 1) // per_round) * per_round

  src = edge_index[0]
  dst = edge_index[1]
  pad = EP - E
  srcp = jnp.concatenate([src, jnp.zeros((pad,), jnp.int32)])
  dstp = jnp.concatenate([dst, jnp.full((pad,), N, jnp.int32)])

  rows_per_tile = NP // NS
  ones128 = jnp.ones((CHUNK, 128), jnp.float32)
  zerosD = jnp.zeros((rows_per_tile, D), jnp.float32)

  deg_kernel = _make_deg_kernel(NP, EP)
  agg_kernel = _make_agg_kernel(N, NP, EP, D)

  deg2 = deg_kernel(dstp, ones128, zerosD)
  dinv, u1 = _tc_pre(deg2, x, W1, N)

  s1 = agg_kernel(u1, srcp, dstp, zerosD)
  u2 = _tc_layer(s1, u1, dinv, b1, g1, be1, W2, N, D)

  s2 = agg_kernel(u2, srcp, dstp, zerosD)
  u3 = _tc_layer(s2, u2, dinv, b2, g2, be2, W3, N, D)

  s3 = agg_kernel(u3, srcp, dstp, zerosD)
  out = _tc_final(s3, u3, dinv, b3, g3, be3, Wout, bout, N)
  return out
